# baseline (speedup 1.0000x reference)
import jax
import jax.numpy as jnp
from jax import lax
from jax.experimental import pallas as pl
from jax.experimental.pallas import tpu as pltpu
from jax.experimental.pallas import tpu_sc as plsc

N_TOK = 32768
B_SEG = 16
LANES = 16
NUM_TILES = 16
CHUNK = N_TOK // NUM_TILES
NV = CHUNK // LANES
MAX_PRUNED = 18


def _sparsemax_body(x_hbm, b_hbm, out_hbm, xv, sv, ov, xk, tau, pc,
                    iota_v, shacc, accl, zv, act, sem1, sem2):
    def sload(ref, idx):
        return ref[pl.ds(idx, LANES)][0]
    wid = lax.axis_index("s")
    base = wid * CHUNK
    cp1 = pltpu.async_copy(x_hbm.at[pl.ds(base, CHUNK)], xv, sem1)
    cp2 = pltpu.async_copy(b_hbm.at[pl.ds(base, CHUNK)], sv, sem2)

    tau[pl.ds(0, LANES)] = jnp.zeros((LANES,), jnp.float32)
    iota_v[pl.ds(0, LANES)] = lax.iota(jnp.int32, LANES)
    iota_v[pl.ds(LANES, LANES)] = lax.iota(jnp.int32, LANES) + LANES
    ones = jnp.ones((LANES,), jnp.float32)
    zeros = jnp.zeros((LANES,), jnp.float32)
    izeros = jnp.zeros((LANES,), jnp.int32)
    lane = lax.iota(jnp.int32, LANES)

    @pl.when(wid == 0)
    def _():
        zv[pl.ds(0, LANES)] = zeros
        zv[pl.ds(LANES, LANES)] = zeros
        pltpu.sync_copy(zv, shacc)

    cp1.wait()
    cp2.wait()
    plsc.subcore_barrier()

    def reduce_and_update(prev_s, prev_c):
        pltpu.sync_copy(pc, shacc.at[iota_v], add=True)
        plsc.subcore_barrier()
        pltpu.sync_copy(shacc, accl)
        s_acc = accl[pl.ds(0, LANES)]
        c_acc = accl[pl.ds(LANES, LANES)]
        tau_new = (s_acc - prev_s - 1.0) / jnp.maximum(c_acc - prev_c, 1.0)
        return s_acc, c_acc, tau_new

    pc[pl.ds(0, LANES)] = zeros
    pc[pl.ds(LANES, LANES)] = zeros

    def scan_totals(i, carry2):
        off = i * LANES
        seg = sv[pl.ds(off, LANES)]
        vx = xv[pl.ds(off, LANES)]
        plsc.addupdate_scatter(pc, [seg], vx)
        plsc.addupdate_scatter(pc, [seg + LANES], ones)
        return carry2

    lax.fori_loop(0, NV, scan_totals, 0, unroll=8)
    s_acc, c_acc, tau_new = reduce_and_update(zeros, zeros)
    tau[pl.ds(0, LANES)] = tau_new
    plsc.subcore_barrier()

    cum_inc = plsc.cumsum(c_acc)
    cum_exc = cum_inc - c_acc
    lo_g = cum_exc.astype(jnp.int32) - base
    hi_g = cum_inc.astype(jnp.int32) - base
    lo_cl = jnp.clip(lo_g, 0, CHUNK)
    hi_cl = jnp.clip(hi_g, 0, CHUNK)
    nonempty = hi_cl > lo_cl
    plsc.store_compressed(act.at[pl.ds(0, LANES)], lane, mask=nonempty)
    plsc.store_compressed(act.at[pl.ds(LANES, LANES)], lo_cl, mask=nonempty)
    plsc.store_compressed(act.at[pl.ds(2 * LANES, LANES)], hi_cl,
                          mask=nonempty)
    n_act = plsc.all_reduce_population_count(nonempty)[0]

    pc[pl.ds(0, LANES)] = zeros
    pc[pl.ds(LANES, LANES)] = zeros

    lane0 = lane == 0

    def range_iter1(j, carry):
        cursor, klo_v, khi_v = carry
        s = sload(act, j)
        lo = sload(act, LANES + j)
        hi = sload(act, 2 * LANES + j)
        tau_b = zeros + sload(tau, s)
        klo_v = jnp.where(lane == j, izeros + cursor, klo_v)

        def inner(k, carry3):
            cur, s_v, c_v = carry3
            off = k * LANES
            vx = xv[pl.ds(off, LANES)]
            gi = off + lane
            m = jnp.logical_and(
                jnp.logical_and(gi >= lo, gi < hi), vx > tau_b)
            plsc.store_compressed(xk.at[pl.ds(cur, LANES)], vx, mask=m)
            s_v = s_v + jnp.where(m, vx, 0.0)
            c_v = c_v + m.astype(jnp.int32)
            return (cur + plsc.all_reduce_population_count(m)[0], s_v, c_v)

        cursor, s_v, c_v = lax.fori_loop(
            lo // LANES, (hi + LANES - 1) // LANES, inner,
            (cursor, zeros, izeros))
        plsc.addupdate_scatter(pc, [izeros + s], zeros + jnp.sum(s_v),
                               mask=lane0)
        plsc.addupdate_scatter(pc, [izeros + (LANES + s)],
                               zeros + jnp.sum(c_v).astype(jnp.float32),
                               mask=lane0)
        khi_v = jnp.where(lane == j, izeros + cursor, khi_v)
        return (cursor, klo_v, khi_v)

    _, klo_v, khi_v = lax.fori_loop(0, n_act, range_iter1,
                                    (0, izeros, izeros))
    act[pl.ds(3 * LANES, LANES)] = klo_v
    act[pl.ds(4 * LANES, LANES)] = khi_v
    s_acc, c_acc, tau_new = reduce_and_update(s_acc, c_acc)
    tau[pl.ds(0, LANES)] = tau_new
    plsc.subcore_barrier()

    def range_pruned(j, carry2):
        s = sload(act, j)
        klo = sload(act, 3 * LANES + j)
        khi = sload(act, 4 * LANES + j)
        tau_b = zeros + sload(tau, s)

        def inner(k, carry3):
            s_v, c_v = carry3
            off = k * LANES
            vx = xk[pl.ds(off, LANES)]
            gi = off + lane
            m = jnp.logical_and(
                jnp.logical_and(gi >= klo, gi < khi), vx > tau_b)
            return (s_v + jnp.where(m, vx, 0.0), c_v + m.astype(jnp.int32))

        s_v, c_v = lax.fori_loop(
            klo // LANES, (khi + LANES - 1) // LANES, inner, (zeros, izeros))
        plsc.addupdate_scatter(pc, [izeros + s], zeros + jnp.sum(s_v),
                               mask=lane0)
        plsc.addupdate_scatter(pc, [izeros + (LANES + s)],
                               zeros + jnp.sum(c_v).astype(jnp.float32),
                               mask=lane0)
        return carry2

    def pruned_cond(carry):
        _, _, t, done = carry
        return jnp.logical_and(t < MAX_PRUNED, jnp.logical_not(done))

    def pruned_iter(carry):
        prev_s2, prev_c2, t, _ = carry
        pc[pl.ds(0, LANES)] = zeros
        pc[pl.ds(LANES, LANES)] = zeros
        lax.fori_loop(0, n_act, range_pruned, 0)
        tau_old = tau[pl.ds(0, LANES)]
        s_acc2, c_acc2, tau_new2 = reduce_and_update(prev_s2, prev_c2)
        done = jnp.logical_not(jnp.any(tau_new2 != tau_old))
        tau[pl.ds(0, LANES)] = tau_new2
        plsc.subcore_barrier()
        return (s_acc2, c_acc2, t + 1, done)

    lax.while_loop(pruned_cond, pruned_iter,
                   (s_acc, c_acc, 0, jnp.bool_(False)))

    def range_out(j, carry2):
        s = sload(act, j)
        lo = sload(act, LANES + j)
        hi = sload(act, 2 * LANES + j)
        tau_b = zeros + sload(tau, s)

        def inner(k, carry3):
            off = k * LANES
            vx = xv[pl.ds(off, LANES)]
            gi = off + lane
            m = jnp.logical_and(gi >= lo, gi < hi)
            res = jnp.maximum(vx - tau_b, 0.0)
            ov[pl.ds(off, LANES)] = jnp.where(m, res, ov[pl.ds(off, LANES)])
            return carry3

        lax.fori_loop(lo // LANES, (hi + LANES - 1) // LANES, inner, 0)
        return carry2

    lax.fori_loop(0, n_act, range_out, 0)
    pltpu.sync_copy(ov, out_hbm.at[pl.ds(base, CHUNK)])


@jax.jit
def _sparsemax_sc(x, batch):
    mesh = plsc.VectorSubcoreMesh(
        core_axis_name="c", subcore_axis_name="s", num_cores=1,
        num_subcores=NUM_TILES,
    )
    return pl.kernel(
        _sparsemax_body,
        out_type=jax.ShapeDtypeStruct((N_TOK,), jnp.float32),
        mesh=mesh,
        compiler_params=pltpu.CompilerParams(needs_layout_passes=False),
        scratch_types=[
            pltpu.VMEM((CHUNK,), jnp.float32),
            pltpu.VMEM((CHUNK,), jnp.int32),
            pltpu.VMEM((CHUNK,), jnp.float32),
            pltpu.VMEM((CHUNK + LANES,), jnp.float32),
            pltpu.VMEM((LANES,), jnp.float32),
            pltpu.VMEM((2 * B_SEG,), jnp.float32),
            pltpu.VMEM((2 * B_SEG,), jnp.int32),
            pltpu.VMEM_SHARED((2 * B_SEG,), jnp.float32),
            pltpu.VMEM((2 * B_SEG,), jnp.float32),
            pltpu.VMEM((2 * B_SEG,), jnp.float32),
            pltpu.VMEM((6 * LANES,), jnp.int32),
            pltpu.SemaphoreType.DMA,
            pltpu.SemaphoreType.DMA,
        ],
    )(x, batch)


def kernel(x, batch):
    return _sparsemax_sc(x, batch.astype(jnp.int32))


# speedup vs baseline: 1.0222x; 1.0222x over previous
import jax
import jax.numpy as jnp
from jax import lax
from jax.experimental import pallas as pl
from jax.experimental.pallas import tpu as pltpu
from jax.experimental.pallas import tpu_sc as plsc

N_TOK = 32768
B_SEG = 16
LANES = 16
NUM_TILES = 16
CHUNK = N_TOK // NUM_TILES
NV = CHUNK // LANES
MAX_PRUNED = 18


def _sparsemax_body(x_hbm, b_hbm, out_hbm, xv, sv, ov, xk, tau, pc,
                    iota_v, shacc, accl, zv, act, shmax, mread, pmaxb,
                    midx, sem1, sem2):
    def sload(ref, idx):
        return ref[pl.ds(idx, LANES)][0]
    wid = lax.axis_index("s")
    base = wid * CHUNK
    cp1 = pltpu.async_copy(x_hbm.at[pl.ds(base, CHUNK)], xv, sem1)
    cp2 = pltpu.async_copy(b_hbm.at[pl.ds(base, CHUNK)], sv, sem2)

    tau[pl.ds(0, LANES)] = jnp.zeros((LANES,), jnp.float32)
    iota_v[pl.ds(0, LANES)] = lax.iota(jnp.int32, LANES)
    iota_v[pl.ds(LANES, LANES)] = lax.iota(jnp.int32, LANES) + LANES
    ones = jnp.ones((LANES,), jnp.float32)
    zeros = jnp.zeros((LANES,), jnp.float32)
    izeros = jnp.zeros((LANES,), jnp.int32)
    lane = lax.iota(jnp.int32, LANES)

    @pl.when(wid == 0)
    def _():
        zv[pl.ds(0, LANES)] = zeros
        zv[pl.ds(LANES, LANES)] = zeros
        pltpu.sync_copy(zv, shacc)
        for _t in range(NUM_TILES):
            mread[pl.ds(_t * LANES, LANES)] = zeros
        pltpu.sync_copy(mread, shmax)

    cp1.wait()
    cp2.wait()
    plsc.subcore_barrier()

    def reduce_and_update(prev_s, prev_c):
        pltpu.sync_copy(pc, shacc.at[iota_v], add=True)
        plsc.subcore_barrier()
        pltpu.sync_copy(shacc, accl)
        s_acc = accl[pl.ds(0, LANES)]
        c_acc = accl[pl.ds(LANES, LANES)]
        tau_new = (s_acc - prev_s - 1.0) / jnp.maximum(c_acc - prev_c, 1.0)
        return s_acc, c_acc, tau_new

    pc[pl.ds(0, LANES)] = zeros
    pc[pl.ds(LANES, LANES)] = zeros

    def scan_totals(i, carry2):
        off = i * LANES
        seg = sv[pl.ds(off, LANES)]
        vx = xv[pl.ds(off, LANES)]
        plsc.addupdate_scatter(pc, [seg], vx)
        plsc.addupdate_scatter(pc, [seg + LANES], ones)
        return carry2

    lax.fori_loop(0, NV, scan_totals, 0, unroll=8)
    s_acc, c_acc, tau_new = reduce_and_update(zeros, zeros)
    tau[pl.ds(0, LANES)] = tau_new
    plsc.subcore_barrier()

    cum_inc = plsc.cumsum(c_acc)
    cum_exc = cum_inc - c_acc
    lo_g = cum_exc.astype(jnp.int32) - base
    hi_g = cum_inc.astype(jnp.int32) - base
    lo_cl = jnp.clip(lo_g, 0, CHUNK)
    hi_cl = jnp.clip(hi_g, 0, CHUNK)
    nonempty = hi_cl > lo_cl
    plsc.store_compressed(act.at[pl.ds(0, LANES)], lane, mask=nonempty)
    plsc.store_compressed(act.at[pl.ds(LANES, LANES)], lo_cl, mask=nonempty)
    plsc.store_compressed(act.at[pl.ds(2 * LANES, LANES)], hi_cl,
                          mask=nonempty)
    n_act = plsc.all_reduce_population_count(nonempty)[0]

    def range_max(j, pmax_v):
        s = sload(act, j)
        lo = sload(act, LANES + j)
        hi = sload(act, 2 * LANES + j)

        def inner(k, mx):
            off = k * LANES
            vx = xv[pl.ds(off, LANES)]
            gi = off + lane
            m = jnp.logical_and(gi >= lo, gi < hi)
            return jnp.maximum(mx, jnp.where(m, vx, -3.0e38))

        mx = lax.fori_loop(lo // LANES, (hi + LANES - 1) // LANES, inner,
                           jnp.full((LANES,), -3.0e38, jnp.float32))
        mxs = jnp.max(mx)
        return jnp.where(lane == s, zeros + (mxs + 1024.0), pmax_v)

    pmaxb[...] = lax.fori_loop(0, n_act, range_max, zeros)
    midx[...] = wid * LANES + lane
    pltpu.sync_copy(pmaxb, shmax.at[midx], add=True)
    plsc.subcore_barrier()
    pltpu.sync_copy(shmax, mread)
    mx_s = mread[pl.ds(0, LANES)]
    for _t in range(1, NUM_TILES):
        mx_s = jnp.maximum(mx_s, mread[pl.ds(_t * LANES, LANES)])
    tau_new = jnp.maximum(tau_new, mx_s - 1025.0)
    tau[pl.ds(0, LANES)] = tau_new

    pc[pl.ds(0, LANES)] = zeros
    pc[pl.ds(LANES, LANES)] = zeros

    lane0 = lane == 0

    def range_iter1(j, carry):
        cursor, klo_v, khi_v = carry
        s = sload(act, j)
        lo = sload(act, LANES + j)
        hi = sload(act, 2 * LANES + j)
        tau_b = zeros + sload(tau, s)
        klo_v = jnp.where(lane == j, izeros + cursor, klo_v)

        def inner(k, carry3):
            cur, s_v, c_v = carry3
            off = k * LANES
            vx = xv[pl.ds(off, LANES)]
            gi = off + lane
            m = jnp.logical_and(
                jnp.logical_and(gi >= lo, gi < hi), vx > tau_b)
            plsc.store_compressed(xk.at[pl.ds(cur, LANES)], vx, mask=m)
            s_v = s_v + jnp.where(m, vx, 0.0)
            c_v = c_v + m.astype(jnp.int32)
            return (cur + plsc.all_reduce_population_count(m)[0], s_v, c_v)

        cursor, s_v, c_v = lax.fori_loop(
            lo // LANES, (hi + LANES - 1) // LANES, inner,
            (cursor, zeros, izeros))
        plsc.addupdate_scatter(pc, [izeros + s], zeros + jnp.sum(s_v),
                               mask=lane0)
        plsc.addupdate_scatter(pc, [izeros + (LANES + s)],
                               zeros + jnp.sum(c_v).astype(jnp.float32),
                               mask=lane0)
        khi_v = jnp.where(lane == j, izeros + cursor, khi_v)
        return (cursor, klo_v, khi_v)

    _, klo_v, khi_v = lax.fori_loop(0, n_act, range_iter1,
                                    (0, izeros, izeros))
    act[pl.ds(3 * LANES, LANES)] = klo_v
    act[pl.ds(4 * LANES, LANES)] = khi_v
    s_acc, c_acc, tau_new = reduce_and_update(s_acc, c_acc)
    tau[pl.ds(0, LANES)] = tau_new
    plsc.subcore_barrier()

    def range_pruned(j, carry2):
        s = sload(act, j)
        klo = sload(act, 3 * LANES + j)
        khi = sload(act, 4 * LANES + j)
        tau_b = zeros + sload(tau, s)

        def inner(k, carry3):
            s_v, c_v = carry3
            off = k * LANES
            vx = xk[pl.ds(off, LANES)]
            gi = off + lane
            m = jnp.logical_and(
                jnp.logical_and(gi >= klo, gi < khi), vx > tau_b)
            return (s_v + jnp.where(m, vx, 0.0), c_v + m.astype(jnp.int32))

        s_v, c_v = lax.fori_loop(
            klo // LANES, (khi + LANES - 1) // LANES, inner, (zeros, izeros))
        plsc.addupdate_scatter(pc, [izeros + s], zeros + jnp.sum(s_v),
                               mask=lane0)
        plsc.addupdate_scatter(pc, [izeros + (LANES + s)],
                               zeros + jnp.sum(c_v).astype(jnp.float32),
                               mask=lane0)
        return carry2

    def pruned_cond(carry):
        _, _, t, done = carry
        return jnp.logical_and(t < MAX_PRUNED, jnp.logical_not(done))

    def pruned_iter(carry):
        prev_s2, prev_c2, t, _ = carry
        pc[pl.ds(0, LANES)] = zeros
        pc[pl.ds(LANES, LANES)] = zeros
        lax.fori_loop(0, n_act, range_pruned, 0)
        tau_old = tau[pl.ds(0, LANES)]
        s_acc2, c_acc2, tau_new2 = reduce_and_update(prev_s2, prev_c2)
        done = jnp.logical_not(jnp.any(tau_new2 != tau_old))
        tau[pl.ds(0, LANES)] = tau_new2
        plsc.subcore_barrier()
        return (s_acc2, c_acc2, t + 1, done)

    lax.while_loop(pruned_cond, pruned_iter,
                   (s_acc, c_acc, 0, jnp.bool_(False)))

    def range_out(j, carry2):
        s = sload(act, j)
        lo = sload(act, LANES + j)
        hi = sload(act, 2 * LANES + j)
        tau_b = zeros + sload(tau, s)

        def inner(k, carry3):
            off = k * LANES
            vx = xv[pl.ds(off, LANES)]
            gi = off + lane
            m = jnp.logical_and(gi >= lo, gi < hi)
            res = jnp.maximum(vx - tau_b, 0.0)
            ov[pl.ds(off, LANES)] = jnp.where(m, res, ov[pl.ds(off, LANES)])
            return carry3

        lax.fori_loop(lo // LANES, (hi + LANES - 1) // LANES, inner, 0)
        return carry2

    lax.fori_loop(0, n_act, range_out, 0)
    pltpu.sync_copy(ov, out_hbm.at[pl.ds(base, CHUNK)])


@jax.jit
def _sparsemax_sc(x, batch):
    mesh = plsc.VectorSubcoreMesh(
        core_axis_name="c", subcore_axis_name="s", num_cores=1,
        num_subcores=NUM_TILES,
    )
    return pl.kernel(
        _sparsemax_body,
        out_type=jax.ShapeDtypeStruct((N_TOK,), jnp.float32),
        mesh=mesh,
        compiler_params=pltpu.CompilerParams(needs_layout_passes=False),
        scratch_types=[
            pltpu.VMEM((CHUNK,), jnp.float32),
            pltpu.VMEM((CHUNK,), jnp.int32),
            pltpu.VMEM((CHUNK,), jnp.float32),
            pltpu.VMEM((CHUNK + LANES,), jnp.float32),
            pltpu.VMEM((LANES,), jnp.float32),
            pltpu.VMEM((2 * B_SEG,), jnp.float32),
            pltpu.VMEM((2 * B_SEG,), jnp.int32),
            pltpu.VMEM_SHARED((2 * B_SEG,), jnp.float32),
            pltpu.VMEM((2 * B_SEG,), jnp.float32),
            pltpu.VMEM((2 * B_SEG,), jnp.float32),
            pltpu.VMEM((6 * LANES,), jnp.int32),
            pltpu.VMEM_SHARED((NUM_TILES * LANES,), jnp.float32),
            pltpu.VMEM((NUM_TILES * LANES,), jnp.float32),
            pltpu.VMEM((LANES,), jnp.float32),
            pltpu.VMEM((LANES,), jnp.int32),
            pltpu.SemaphoreType.DMA,
            pltpu.SemaphoreType.DMA,
        ],
    )(x, batch)


def kernel(x, batch):
    return _sparsemax_sc(x, batch.astype(jnp.int32))


# speedup vs baseline: 1.1796x; 1.1540x over previous
import jax
import jax.numpy as jnp
from jax import lax
from jax.experimental import pallas as pl
from jax.experimental.pallas import tpu as pltpu
from jax.experimental.pallas import tpu_sc as plsc

N_TOK = 32768
B_SEG = 16
LANES = 16
NUM_TILES = 16
CHUNK = N_TOK // NUM_TILES
NV = CHUNK // LANES
MAX_PRUNED = 18


def _sparsemax_body(x_hbm, b_hbm, out_hbm, xv, sv, ov, xk, tau, pc,
                    iota_v, shacc, accl, zv, act, shmax, mread, pmaxb,
                    midx, sem1, sem2):
    def sload(ref, idx):
        return ref[pl.ds(idx, LANES)][0]
    wid = lax.axis_index("s")
    base = wid * CHUNK
    cp1 = pltpu.async_copy(x_hbm.at[pl.ds(base, CHUNK)], xv, sem1)
    cp2 = pltpu.async_copy(b_hbm.at[pl.ds(base, CHUNK)],
                           sv.at[pl.ds(0, CHUNK)], sem2)

    tau[pl.ds(0, LANES)] = jnp.zeros((LANES,), jnp.float32)
    iota_v[pl.ds(0, LANES)] = lax.iota(jnp.int32, LANES)
    iota_v[pl.ds(LANES, LANES)] = lax.iota(jnp.int32, LANES) + LANES
    ones = jnp.ones((LANES,), jnp.float32)
    zeros = jnp.zeros((LANES,), jnp.float32)
    izeros = jnp.zeros((LANES,), jnp.int32)
    lane = lax.iota(jnp.int32, LANES)

    @pl.when(wid == 0)
    def _():
        zv[pl.ds(0, LANES)] = zeros
        zv[pl.ds(LANES, LANES)] = zeros
        pltpu.sync_copy(zv, shacc)
        for _t in range(NUM_TILES):
            mread[pl.ds(_t * LANES, LANES)] = zeros
        pltpu.sync_copy(mread, shmax)

    cp1.wait()
    cp2.wait()

    @pl.when(wid == NUM_TILES - 1)
    def _():
        sv[pl.ds(CHUNK, LANES)] = jnp.full((LANES,), B_SEG, jnp.int32)

    @pl.when(wid < NUM_TILES - 1)
    def _():
        pltpu.sync_copy(b_hbm.at[pl.ds(base + CHUNK, LANES)],
                        sv.at[pl.ds(CHUNK, LANES)])

    plsc.subcore_barrier()

    def reduce_and_update(prev_s, prev_c):
        pltpu.sync_copy(pc, shacc.at[iota_v], add=True)
        plsc.subcore_barrier()
        pltpu.sync_copy(shacc, accl)
        s_acc = accl[pl.ds(0, LANES)]
        c_acc = accl[pl.ds(LANES, LANES)]
        tau_new = (s_acc - prev_s - 1.0) / jnp.maximum(c_acc - prev_c, 1.0)
        return s_acc, c_acc, tau_new

    pc[pl.ds(0, LANES)] = zeros
    pc[pl.ds(LANES, LANES)] = zeros

    def scan_bounds(i, carry2):
        off = i * LANES
        seg = sv[pl.ds(off, LANES)]
        segn = sv[pl.ds(off + 1, LANES)]
        mb = seg != segn
        pos = (base + off + 1) + lane
        plsc.addupdate_scatter(pc, [seg + LANES], pos.astype(jnp.float32),
                               mask=mb)
        return carry2

    lax.fori_loop(0, NV, scan_bounds, 0, unroll=8)
    s_acc, c_acc, _ = reduce_and_update(zeros, zeros)
    plsc.subcore_barrier()

    cum_inc = plsc.cummax(c_acc)
    zv[pl.ds(0, LANES)] = zeros
    zv[pl.ds(1, LANES)] = cum_inc
    cum_exc = zv[pl.ds(0, LANES)]
    lo_g = cum_exc.astype(jnp.int32) - base
    hi_g = cum_inc.astype(jnp.int32) - base
    lo_cl = jnp.clip(lo_g, 0, CHUNK)
    hi_cl = jnp.clip(hi_g, 0, CHUNK)
    nonempty = hi_cl > lo_cl
    plsc.store_compressed(act.at[pl.ds(0, LANES)], lane, mask=nonempty)
    plsc.store_compressed(act.at[pl.ds(LANES, LANES)], lo_cl, mask=nonempty)
    plsc.store_compressed(act.at[pl.ds(2 * LANES, LANES)], hi_cl,
                          mask=nonempty)
    n_act = plsc.all_reduce_population_count(nonempty)[0]

    def range_max(j, pmax_v):
        s = sload(act, j)
        lo = sload(act, LANES + j)
        hi = sload(act, 2 * LANES + j)

        def inner(k, mx):
            off = k * LANES
            vx = xv[pl.ds(off, LANES)]
            gi = off + lane
            m = jnp.logical_and(gi >= lo, gi < hi)
            return jnp.maximum(mx, jnp.where(m, vx, -3.0e38))

        mx = lax.fori_loop(lo // LANES, (hi + LANES - 1) // LANES, inner,
                           jnp.full((LANES,), -3.0e38, jnp.float32))
        mxs = jnp.max(mx)
        return jnp.where(lane == s, zeros + (mxs + 1024.0), pmax_v)

    pmaxb[...] = lax.fori_loop(0, n_act, range_max, zeros)
    midx[...] = wid * LANES + lane
    pltpu.sync_copy(pmaxb, shmax.at[midx], add=True)
    plsc.subcore_barrier()
    pltpu.sync_copy(shmax, mread)
    mx_s = mread[pl.ds(0, LANES)]
    for _t in range(1, NUM_TILES):
        mx_s = jnp.maximum(mx_s, mread[pl.ds(_t * LANES, LANES)])
    tau_new = mx_s - 1025.0
    tau[pl.ds(0, LANES)] = tau_new

    pc[pl.ds(0, LANES)] = zeros
    pc[pl.ds(LANES, LANES)] = zeros

    lane0 = lane == 0

    def range_iter1(j, carry):
        cursor, klo_v, khi_v = carry
        s = sload(act, j)
        lo = sload(act, LANES + j)
        hi = sload(act, 2 * LANES + j)
        tau_b = zeros + sload(tau, s)
        klo_v = jnp.where(lane == j, izeros + cursor, klo_v)

        def inner(k, carry3):
            cur, s_v, c_v = carry3
            off = k * LANES
            vx = xv[pl.ds(off, LANES)]
            gi = off + lane
            m = jnp.logical_and(
                jnp.logical_and(gi >= lo, gi < hi), vx > tau_b)
            plsc.store_compressed(xk.at[pl.ds(cur, LANES)], vx, mask=m)
            s_v = s_v + jnp.where(m, vx, 0.0)
            c_v = c_v + m.astype(jnp.int32)
            return (cur + plsc.all_reduce_population_count(m)[0], s_v, c_v)

        cursor, s_v, c_v = lax.fori_loop(
            lo // LANES, (hi + LANES - 1) // LANES, inner,
            (cursor, zeros, izeros))
        plsc.addupdate_scatter(pc, [izeros + s], zeros + jnp.sum(s_v),
                               mask=lane0)
        plsc.addupdate_scatter(pc, [izeros + (LANES + s)],
                               zeros + jnp.sum(c_v).astype(jnp.float32),
                               mask=lane0)
        khi_v = jnp.where(lane == j, izeros + cursor, khi_v)
        return (cursor, klo_v, khi_v)

    _, klo_v, khi_v = lax.fori_loop(0, n_act, range_iter1,
                                    (0, izeros, izeros))
    act[pl.ds(3 * LANES, LANES)] = klo_v
    act[pl.ds(4 * LANES, LANES)] = khi_v
    s_acc, c_acc, tau_new = reduce_and_update(s_acc, c_acc)
    tau[pl.ds(0, LANES)] = tau_new
    plsc.subcore_barrier()

    def range_pruned(j, carry2):
        s = sload(act, j)
        klo = sload(act, 3 * LANES + j)
        khi = sload(act, 4 * LANES + j)
        tau_b = zeros + sload(tau, s)

        def inner(k, carry3):
            s_v, c_v = carry3
            off = k * LANES
            vx = xk[pl.ds(off, LANES)]
            gi = off + lane
            m = jnp.logical_and(
                jnp.logical_and(gi >= klo, gi < khi), vx > tau_b)
            return (s_v + jnp.where(m, vx, 0.0), c_v + m.astype(jnp.int32))

        s_v, c_v = lax.fori_loop(
            klo // LANES, (khi + LANES - 1) // LANES, inner, (zeros, izeros))
        plsc.addupdate_scatter(pc, [izeros + s], zeros + jnp.sum(s_v),
                               mask=lane0)
        plsc.addupdate_scatter(pc, [izeros + (LANES + s)],
                               zeros + jnp.sum(c_v).astype(jnp.float32),
                               mask=lane0)
        return carry2

    def pruned_cond(carry):
        _, _, t, done = carry
        return jnp.logical_and(t < MAX_PRUNED, jnp.logical_not(done))

    def pruned_iter(carry):
        prev_s2, prev_c2, t, _ = carry
        pc[pl.ds(0, LANES)] = zeros
        pc[pl.ds(LANES, LANES)] = zeros
        lax.fori_loop(0, n_act, range_pruned, 0)
        tau_old = tau[pl.ds(0, LANES)]
        s_acc2, c_acc2, tau_new2 = reduce_and_update(prev_s2, prev_c2)
        done = jnp.logical_not(jnp.any(tau_new2 != tau_old))
        tau[pl.ds(0, LANES)] = tau_new2
        plsc.subcore_barrier()
        return (s_acc2, c_acc2, t + 1, done)

    lax.while_loop(pruned_cond, pruned_iter,
                   (s_acc, c_acc, 0, jnp.bool_(False)))

    def range_out(j, carry2):
        s = sload(act, j)
        lo = sload(act, LANES + j)
        hi = sload(act, 2 * LANES + j)
        tau_b = zeros + sload(tau, s)

        def inner(k, carry3):
            off = k * LANES
            vx = xv[pl.ds(off, LANES)]
            gi = off + lane
            m = jnp.logical_and(gi >= lo, gi < hi)
            res = jnp.maximum(vx - tau_b, 0.0)
            ov[pl.ds(off, LANES)] = jnp.where(m, res, ov[pl.ds(off, LANES)])
            return carry3

        lax.fori_loop(lo // LANES, (hi + LANES - 1) // LANES, inner, 0)
        return carry2

    lax.fori_loop(0, n_act, range_out, 0)
    pltpu.sync_copy(ov, out_hbm.at[pl.ds(base, CHUNK)])


@jax.jit
def _sparsemax_sc(x, batch):
    mesh = plsc.VectorSubcoreMesh(
        core_axis_name="c", subcore_axis_name="s", num_cores=1,
        num_subcores=NUM_TILES,
    )
    return pl.kernel(
        _sparsemax_body,
        out_type=jax.ShapeDtypeStruct((N_TOK,), jnp.float32),
        mesh=mesh,
        compiler_params=pltpu.CompilerParams(needs_layout_passes=False),
        scratch_types=[
            pltpu.VMEM((CHUNK,), jnp.float32),
            pltpu.VMEM((CHUNK + LANES,), jnp.int32),
            pltpu.VMEM((CHUNK,), jnp.float32),
            pltpu.VMEM((CHUNK + LANES,), jnp.float32),
            pltpu.VMEM((LANES,), jnp.float32),
            pltpu.VMEM((2 * B_SEG,), jnp.float32),
            pltpu.VMEM((2 * B_SEG,), jnp.int32),
            pltpu.VMEM_SHARED((2 * B_SEG,), jnp.float32),
            pltpu.VMEM((2 * B_SEG,), jnp.float32),
            pltpu.VMEM((2 * B_SEG,), jnp.float32),
            pltpu.VMEM((6 * LANES,), jnp.int32),
            pltpu.VMEM_SHARED((NUM_TILES * LANES,), jnp.float32),
            pltpu.VMEM((NUM_TILES * LANES,), jnp.float32),
            pltpu.VMEM((LANES,), jnp.float32),
            pltpu.VMEM((LANES,), jnp.int32),
            pltpu.SemaphoreType.DMA,
            pltpu.SemaphoreType.DMA,
        ],
    )(x, batch)


def kernel(x, batch):
    return _sparsemax_sc(x, batch.astype(jnp.int32))
